# baseline (speedup 1.0000x reference)
import jax
import jax.numpy as jnp
from jax.experimental import pallas as pl
from jax.experimental.pallas import tpu as pltpu


def _fused_kernel(x_ref, Wz_ref, bz_ref, Wh_ref, bh_ref,
                  Lz_ref, lzb_ref, Lh_ref, lhb_ref,
                  att_ref, linW_ref, linb_ref, out_ref,
                  gz_acc, gh_acc):
    k = pl.program_id(0)
    dn = (((0,), (0,)), ((), ()))
    pz = jax.lax.dot_general(x_ref[:], Wz_ref[:], dn,
                             preferred_element_type=jnp.float32)
    ph = jax.lax.dot_general(x_ref[:], Wh_ref[:], dn,
                             preferred_element_type=jnp.float32)

    @pl.when(k == 0)
    def _init():
        gz_acc[:] = pz
        gh_acc[:] = ph

    @pl.when(k > 0)
    def _accum():
        gz_acc[:] += pz
        gh_acc[:] += ph

    @pl.when(k == pl.num_programs(0) - 1)
    def _tail():
        nf = Wz_ref.shape[1]
        gz = gz_acc[:] + bz_ref[:]
        gh = gh_acc[:] + bh_ref[:]
        az = jnp.dot(gz, Lz_ref[0:nf, :],
                     preferred_element_type=jnp.float32) + lzb_ref[:]
        ah = jnp.dot(gh, Lh_ref[0:nf, :],
                     preferred_element_type=jnp.float32) + lhb_ref[:]
        z = jax.nn.sigmoid(az)
        hn = (1.0 - z) * jnp.tanh(ah)
        att = att_ref[:]
        probs = jnp.exp(att - jnp.max(att, axis=0, keepdims=True))
        probs = probs / jnp.sum(probs, axis=0, keepdims=True)
        hacc = jnp.sum(probs * hn, axis=0, keepdims=True)
        h = jnp.maximum(hacc, 0.0)
        out_ref[:] = (jnp.dot(h, linW_ref[:],
                              preferred_element_type=jnp.float32)
                      + linb_ref[:])


def kernel(x, edge_index, edge_weight, W_z, b_z, W_r, b_r, W_h, b_h,
           lz_W, lz_b, lr_W, lr_b, lh_W, lh_b, att, lin_W, lin_b):
    n, p = x.shape
    nf = W_z.shape[1]
    n_blocks = 4
    blk = n // n_blocks
    full = lambda a: pl.BlockSpec(a.shape, lambda k: (0,) * a.ndim)
    out = pl.pallas_call(
        _fused_kernel,
        grid=(n_blocks,),
        in_specs=[
            pl.BlockSpec((blk, p), lambda k: (k, 0)),
            pl.BlockSpec((blk, nf), lambda k: (k, 0)),
            full(b_z.reshape(1, -1)),
            pl.BlockSpec((blk, nf), lambda k: (k, 0)),
            full(b_h.reshape(1, -1)),
            full(lz_W), full(lz_b.reshape(1, -1)),
            full(lh_W), full(lh_b.reshape(1, -1)),
            full(att.reshape(-1, 1)),
            full(lin_W), full(lin_b.reshape(1, -1)),
        ],
        out_specs=pl.BlockSpec((1, lin_W.shape[1]), lambda k: (0, 0)),
        out_shape=jax.ShapeDtypeStruct((1, lin_W.shape[1]), x.dtype),
        scratch_shapes=[pltpu.VMEM((p, nf), jnp.float32),
                        pltpu.VMEM((p, nf), jnp.float32)],
        compiler_params=pltpu.CompilerParams(
            dimension_semantics=("arbitrary",)),
    )(x, W_z, b_z.reshape(1, -1), W_h, b_h.reshape(1, -1),
      lz_W, lz_b.reshape(1, -1), lh_W, lh_b.reshape(1, -1),
      att.reshape(-1, 1), lin_W, lin_b.reshape(1, -1))
    return (out,)


# speedup vs baseline: 1.1534x; 1.1534x over previous
import jax
import jax.numpy as jnp
from jax.experimental import pallas as pl
from jax.experimental.pallas import tpu as pltpu


def _fused_kernel(x_ref, Wz_ref, bz_ref, Wh_ref, bh_ref,
                  Lz_ref, lzb_ref, Lh_ref, lhb_ref,
                  att_ref, linW_ref, linb_ref, out_ref,
                  gz_acc, gh_acc):
    k = pl.program_id(0)
    dn = (((0,), (0,)), ((), ()))
    pz = jax.lax.dot_general(x_ref[:], Wz_ref[:], dn,
                             preferred_element_type=jnp.float32)
    ph = jax.lax.dot_general(x_ref[:], Wh_ref[:], dn,
                             preferred_element_type=jnp.float32)

    @pl.when(k == 0)
    def _init():
        gz_acc[:] = pz
        gh_acc[:] = ph

    @pl.when(k > 0)
    def _accum():
        gz_acc[:] += pz
        gh_acc[:] += ph

    @pl.when(k == pl.num_programs(0) - 1)
    def _tail():
        nf = Wz_ref.shape[1]
        gz = gz_acc[:] + bz_ref[:]
        gh = gh_acc[:] + bh_ref[:]
        az = jnp.dot(gz, Lz_ref[0:nf, :],
                     preferred_element_type=jnp.float32) + lzb_ref[:]
        ah = jnp.dot(gh, Lh_ref[0:nf, :],
                     preferred_element_type=jnp.float32) + lhb_ref[:]
        z = jax.nn.sigmoid(az)
        hn = (1.0 - z) * jnp.tanh(ah)
        att = att_ref[:]
        probs = jnp.exp(att - jnp.max(att, axis=0, keepdims=True))
        probs = probs / jnp.sum(probs, axis=0, keepdims=True)
        hacc = jnp.sum(probs * hn, axis=0, keepdims=True)
        h = jnp.maximum(hacc, 0.0)
        out_ref[:] = (jnp.dot(h, linW_ref[:],
                              preferred_element_type=jnp.float32)
                      + linb_ref[:])


def kernel(x, edge_index, edge_weight, W_z, b_z, W_r, b_r, W_h, b_h,
           lz_W, lz_b, lr_W, lr_b, lh_W, lh_b, att, lin_W, lin_b):
    n, p = x.shape
    nf = W_z.shape[1]
    n_blocks = 2
    blk = n // n_blocks
    full = lambda a: pl.BlockSpec(a.shape, lambda k: (0,) * a.ndim)
    out = pl.pallas_call(
        _fused_kernel,
        grid=(n_blocks,),
        in_specs=[
            pl.BlockSpec((blk, p), lambda k: (k, 0)),
            pl.BlockSpec((blk, nf), lambda k: (k, 0)),
            full(b_z.reshape(1, -1)),
            pl.BlockSpec((blk, nf), lambda k: (k, 0)),
            full(b_h.reshape(1, -1)),
            full(lz_W), full(lz_b.reshape(1, -1)),
            full(lh_W), full(lh_b.reshape(1, -1)),
            full(att.reshape(-1, 1)),
            full(lin_W), full(lin_b.reshape(1, -1)),
        ],
        out_specs=pl.BlockSpec((1, lin_W.shape[1]), lambda k: (0, 0)),
        out_shape=jax.ShapeDtypeStruct((1, lin_W.shape[1]), x.dtype),
        scratch_shapes=[pltpu.VMEM((p, nf), jnp.float32),
                        pltpu.VMEM((p, nf), jnp.float32)],
        compiler_params=pltpu.CompilerParams(
            dimension_semantics=("arbitrary",)),
    )(x, W_z, b_z.reshape(1, -1), W_h, b_h.reshape(1, -1),
      lz_W, lz_b.reshape(1, -1), lh_W, lh_b.reshape(1, -1),
      att.reshape(-1, 1), lin_W, lin_b.reshape(1, -1))
    return (out,)


# speedup vs baseline: 1.1658x; 1.0108x over previous
import jax
import jax.numpy as jnp
from jax.experimental import pallas as pl
from jax.experimental.pallas import tpu as pltpu


def _fused_kernel(x_ref, Wz_ref, bz_ref, Wh_ref, bh_ref,
                  Lz_ref, lzb_ref, Lh_ref, lhb_ref,
                  att_ref, linW_ref, linb_ref, out_ref,
                  gz_acc, gh_acc):
    k = pl.program_id(0)
    dn = (((0,), (0,)), ((), ()))
    pz = jax.lax.dot_general(x_ref[:], Wz_ref[:], dn,
                             preferred_element_type=jnp.float32)
    ph = jax.lax.dot_general(x_ref[:], Wh_ref[:], dn,
                             preferred_element_type=jnp.float32)

    @pl.when(k == 0)
    def _init():
        gz_acc[:] = pz
        gh_acc[:] = ph

    @pl.when(k > 0)
    def _accum():
        gz_acc[:] += pz
        gh_acc[:] += ph

    @pl.when(k == pl.num_programs(0) - 1)
    def _tail():
        gz = gz_acc[:] + bz_ref[:]
        gh = gh_acc[:] + bh_ref[:]
        az = jnp.dot(gz, Lz_ref[:],
                     preferred_element_type=jnp.float32) + lzb_ref[:]
        ah = jnp.dot(gh, Lh_ref[:],
                     preferred_element_type=jnp.float32) + lhb_ref[:]
        z = jax.nn.sigmoid(az)
        hn = (1.0 - z) * jnp.tanh(ah)
        att = att_ref[:]
        probs = jnp.exp(att - jnp.max(att, axis=0, keepdims=True))
        probs = probs / jnp.sum(probs, axis=0, keepdims=True)
        hacc = jnp.sum(probs * hn, axis=0, keepdims=True)
        h = jnp.maximum(hacc, 0.0)
        out_ref[:] = (jnp.dot(h, linW_ref[:],
                              preferred_element_type=jnp.float32)
                      + linb_ref[:])


def kernel(x, edge_index, edge_weight, W_z, b_z, W_r, b_r, W_h, b_h,
           lz_W, lz_b, lr_W, lr_b, lh_W, lh_b, att, lin_W, lin_b):
    n, p = x.shape
    nf = W_z.shape[1]
    n_blocks = 2
    blk = n // n_blocks
    full = lambda a: pl.BlockSpec(a.shape, lambda k: (0,) * a.ndim)
    out = pl.pallas_call(
        _fused_kernel,
        grid=(n_blocks,),
        in_specs=[
            pl.BlockSpec((blk, p), lambda k: (k, 0)),
            pl.BlockSpec((blk, nf), lambda k: (k, 0)),
            full(b_z.reshape(1, -1)),
            pl.BlockSpec((blk, nf), lambda k: (k, 0)),
            full(b_h.reshape(1, -1)),
            pl.BlockSpec((nf, nf), lambda k: (0, 0)),
            full(lz_b.reshape(1, -1)),
            pl.BlockSpec((nf, nf), lambda k: (0, 0)),
            full(lh_b.reshape(1, -1)),
            full(att.reshape(-1, 1)),
            full(lin_W), full(lin_b.reshape(1, -1)),
        ],
        out_specs=pl.BlockSpec((1, lin_W.shape[1]), lambda k: (0, 0)),
        out_shape=jax.ShapeDtypeStruct((1, lin_W.shape[1]), x.dtype),
        scratch_shapes=[pltpu.VMEM((p, nf), jnp.float32),
                        pltpu.VMEM((p, nf), jnp.float32)],
        compiler_params=pltpu.CompilerParams(
            dimension_semantics=("arbitrary",)),
    )(x, W_z, b_z.reshape(1, -1), W_h, b_h.reshape(1, -1),
      lz_W, lz_b.reshape(1, -1), lh_W, lh_b.reshape(1, -1),
      att.reshape(-1, 1), lin_W, lin_b.reshape(1, -1))
    return (out,)
